# Initial kernel scaffold; baseline (speedup 1.0000x reference)
import jax
import jax.numpy as jnp
from jax.experimental import pallas as pl


def kernel(label, user_id, item_id, category, brand, W_user, W_item, W_category, W_brand):
    raise NotImplementedError("write your pallas kernel here")


# speedup vs baseline: 3.2868x; 3.2868x over previous
import functools

import jax
import jax.numpy as jnp
from jax import lax
from jax.experimental import pallas as pl
from jax.experimental.pallas import tpu as pltpu
from jax.experimental.pallas import tpu_sc as plsc

B = 16384
L = 50
D = 16
NC = 2
NS = 16
NW = NC * NS
BPW = B // NW
GRP = BPW // 16


def _emb_body(user_id, item_id, category, brand,
              w_user, w_item, w_category, w_brand, out_hbm,
              uidx, iidx, bidx, rawcat, cols, urows, irows, brows,
              acc, n0, w0, stage,
              sem_u, sem_i, sem_b, sem_c):
    wid = lax.axis_index("s") * NC + lax.axis_index("c")
    base = wid * BPW

    pltpu.sync_copy(user_id.at[pl.ds(base, BPW)], uidx)
    pltpu.sync_copy(item_id.at[pl.ds(base, BPW)], iidx)
    pltpu.sync_copy(brand.at[pl.ds(base, BPW)], bidx)
    pltpu.sync_copy(category.at[pl.ds(base * L, BPW * L)], rawcat)

    du = pltpu.async_copy(w_user.at[uidx], urows, sem_u)
    di = pltpu.async_copy(w_item.at[iidx], irows, sem_i)
    db = pltpu.async_copy(w_brand.at[bidx], brows, sem_b)

    lane = jnp.arange(16, dtype=jnp.int32)

    lane_l = lane * L

    def g_body(g, _):
        flat16 = lane_l + g * (16 * L)

        def l_body(l, cnt):
            vals = plsc.load_gather(rawcat, [flat16 + l])
            cols[l, pl.ds(g * 16, 16)] = vals
            return cnt + jnp.where(vals == 0, 1, 0).astype(jnp.int32)

        cnt = lax.fori_loop(0, L, l_body, jnp.zeros((16,), jnp.int32))
        n0[pl.ds(g * 16, 16)] = cnt.astype(jnp.float32)
        return 0

    lax.fori_loop(0, GRP, g_body, 0)

    pltpu.async_copy(w_category.at[cols.at[0]], acc, sem_c).wait()

    def c_body(l, _):
        pltpu.async_copy(w_category.at[cols.at[l]], acc, sem_c, add=True).wait()
        return 0

    lax.fori_loop(1, L, c_body, 0)

    pltpu.sync_copy(w_category.at[pl.ds(0, 1)], w0)

    du.wait()
    di.wait()
    db.wait()

    inv_l = jnp.float32(1.0 / L)
    w0s = w0[0, :] * inv_l

    def o_body(b, _):
        stage[b, 0, :] = urows[b, :]
        stage[b, 1, :] = irows[b, :]
        n0b = plsc.load_gather(n0, [jnp.full((16,), b, jnp.int32)])
        stage[b, 2, :] = acc[b, :] * inv_l - n0b * w0s
        stage[b, 3, :] = brows[b, :]
        return 0

    lax.fori_loop(0, BPW, o_body, 0)

    pltpu.sync_copy(stage, out_hbm.at[pl.ds(base, BPW)])


@jax.jit
def _emb(user_id, item_id, category, brand, w_user, w_item, w_category, w_brand):
    mesh = plsc.VectorSubcoreMesh(core_axis_name="c", subcore_axis_name="s")
    f = pl.kernel(
        _emb_body,
        out_type=jax.ShapeDtypeStruct((B, 4, D), jnp.float32),
        mesh=mesh,
        compiler_params=pltpu.CompilerParams(
            needs_layout_passes=False, use_tc_tiling_on_sc=False),
        scratch_types=[
            pltpu.VMEM((BPW,), jnp.int32),
            pltpu.VMEM((BPW,), jnp.int32),
            pltpu.VMEM((BPW,), jnp.int32),
            pltpu.VMEM((BPW * L,), jnp.int32),
            pltpu.VMEM((L, BPW), jnp.int32),
            pltpu.VMEM((BPW, D), jnp.float32),
            pltpu.VMEM((BPW, D), jnp.float32),
            pltpu.VMEM((BPW, D), jnp.float32),
            pltpu.VMEM((BPW, D), jnp.float32),
            pltpu.VMEM((BPW,), jnp.float32),
            pltpu.VMEM((1, D), jnp.float32),
            pltpu.VMEM((BPW, 4, D), jnp.float32),
            pltpu.SemaphoreType.DMA,
            pltpu.SemaphoreType.DMA,
            pltpu.SemaphoreType.DMA,
            pltpu.SemaphoreType.DMA,
        ],
    )
    return f(user_id, item_id, category.reshape(B * L), brand,
             w_user, w_item, w_category, w_brand)


def kernel(label, user_id, item_id, category, brand,
           W_user, W_item, W_category, W_brand):
    del label
    return _emb(user_id, item_id, category, brand,
                W_user, W_item, W_category, W_brand)


# speedup vs baseline: 3.4005x; 1.0346x over previous
import functools

import jax
import jax.numpy as jnp
from jax import lax
from jax.experimental import pallas as pl
from jax.experimental.pallas import tpu as pltpu
from jax.experimental.pallas import tpu_sc as plsc

B = 16384
L = 50
D = 16
NC = 2
NS = 16
NW = NC * NS
BPW = B // NW
GRP = BPW // 16


def _emb_body(user_id, item_id, category, brand,
              w_user, w_item, w_category, w_brand, out_hbm,
              uidx, iidx, bidx, rawcat, cols, urows, irows, brows,
              acc, n0, w0, stage,
              sem_u, sem_i, sem_b, sem_c):
    wid = lax.axis_index("s") * NC + lax.axis_index("c")
    base = wid * BPW

    pltpu.sync_copy(user_id.at[pl.ds(base, BPW)], uidx)
    pltpu.sync_copy(item_id.at[pl.ds(base, BPW)], iidx)
    pltpu.sync_copy(brand.at[pl.ds(base, BPW)], bidx)
    pltpu.sync_copy(category.at[pl.ds(base * L, BPW * L)], rawcat)

    du = pltpu.async_copy(w_user.at[uidx], urows, sem_u)
    di = pltpu.async_copy(w_item.at[iidx], irows, sem_i)
    db = pltpu.async_copy(w_brand.at[bidx], brows, sem_b)

    lane = jnp.arange(16, dtype=jnp.int32)

    lane_l = lane * L

    def g_body(g, _):
        flat16 = lane_l + g * (16 * L)

        def l_body(l, cnt):
            vals = plsc.load_gather(rawcat, [flat16 + l])
            cols[l, pl.ds(g * 16, 16)] = vals
            return cnt + jnp.where(vals == 0, 1, 0).astype(jnp.int32)

        cnt = lax.fori_loop(0, L, l_body, jnp.zeros((16,), jnp.int32))
        n0[pl.ds(g * 16, 16)] = cnt.astype(jnp.float32)
        return 0

    lax.fori_loop(0, GRP, g_body, 0)

    pltpu.async_copy(w_category.at[cols.at[0]], acc, sem_c).wait()

    def c_fire(l, _):
        pltpu.async_copy(w_category.at[cols.at[l]], acc, sem_c, add=True)
        return 0

    lax.fori_loop(1, L, c_fire, 0)

    def c_drain(l, _):
        pltpu.make_async_copy(w_category.at[cols.at[0]], acc, sem_c).wait()
        return 0

    lax.fori_loop(1, L, c_drain, 0)

    pltpu.sync_copy(w_category.at[pl.ds(0, 1)], w0)

    du.wait()
    di.wait()
    db.wait()

    inv_l = jnp.float32(1.0 / L)
    w0s = w0[0, :] * inv_l

    def o_body(b, _):
        stage[b, 0, :] = urows[b, :]
        stage[b, 1, :] = irows[b, :]
        n0b = plsc.load_gather(n0, [jnp.full((16,), b, jnp.int32)])
        stage[b, 2, :] = acc[b, :] * inv_l - n0b * w0s
        stage[b, 3, :] = brows[b, :]
        return 0

    lax.fori_loop(0, BPW, o_body, 0)

    pltpu.sync_copy(stage, out_hbm.at[pl.ds(base, BPW)])


@jax.jit
def _emb(user_id, item_id, category, brand, w_user, w_item, w_category, w_brand):
    mesh = plsc.VectorSubcoreMesh(core_axis_name="c", subcore_axis_name="s")
    f = pl.kernel(
        _emb_body,
        out_type=jax.ShapeDtypeStruct((B, 4, D), jnp.float32),
        mesh=mesh,
        compiler_params=pltpu.CompilerParams(
            needs_layout_passes=False, use_tc_tiling_on_sc=False),
        scratch_types=[
            pltpu.VMEM((BPW,), jnp.int32),
            pltpu.VMEM((BPW,), jnp.int32),
            pltpu.VMEM((BPW,), jnp.int32),
            pltpu.VMEM((BPW * L,), jnp.int32),
            pltpu.VMEM((L, BPW), jnp.int32),
            pltpu.VMEM((BPW, D), jnp.float32),
            pltpu.VMEM((BPW, D), jnp.float32),
            pltpu.VMEM((BPW, D), jnp.float32),
            pltpu.VMEM((BPW, D), jnp.float32),
            pltpu.VMEM((BPW,), jnp.float32),
            pltpu.VMEM((1, D), jnp.float32),
            pltpu.VMEM((BPW, 4, D), jnp.float32),
            pltpu.SemaphoreType.DMA,
            pltpu.SemaphoreType.DMA,
            pltpu.SemaphoreType.DMA,
            pltpu.SemaphoreType.DMA,
        ],
    )
    return f(user_id, item_id, category.reshape(B * L), brand,
             w_user, w_item, w_category, w_brand)


def kernel(label, user_id, item_id, category, brand,
           W_user, W_item, W_category, W_brand):
    del label
    return _emb(user_id, item_id, category, brand,
                W_user, W_item, W_category, W_brand)
